# Initial kernel scaffold; baseline (speedup 1.0000x reference)
import jax
import jax.numpy as jnp
from jax.experimental import pallas as pl


def kernel(x, edge_index, batch, edge_attr, Wg0, Wr0, b0, mu0, sg0, Wg1, Wr1, b1, mu1, sg1, Wg2, Wr2, b2, mu2, sg2, Wg3, Wr3, b3, mu3, sg3, mW1, mb1, bn_g, bn_b, bn_m, bn_v, mW2, mb2):
    raise NotImplementedError("write your pallas kernel here")


# speedup vs baseline: 2.0086x; 2.0086x over previous
import functools

import jax
import jax.numpy as jnp
from jax import lax
from jax.experimental import pallas as pl
from jax.experimental.pallas import tpu as pltpu
from jax.experimental.pallas import tpu_sc as plsc

N = 10000
E = 160000
K = 4
D = 4
H = 64
F_IN = 128
C = 10
G = 64
EPS = 1e-15
ALPHA = 0.01

CHUNK = 128
NW = 32
NCH = 1280
E_PAD = NCH * CHUNK
CPW = NCH // NW
RPT = 632
RPT_LAST = N - 15 * RPT
AW = 128


def _leaky(v):
    return jnp.where(v >= 0, v, ALPHA * v)


def _gw_body(mu_ref, iv_ref, ea_ref, val_ref, out_ref):
    val = val_ref[...]
    for l in range(4):
        for k in range(K):
            acc = jnp.zeros_like(val)
            for d in range(D):
                t = ea_ref[d] - mu_ref[l, k, d]
                acc = acc + t * t * iv_ref[l, k, d]
            out_ref[l, :, k, :] = jnp.exp(-0.5 * acc) * val
        out_ref[l, :, 4, :] = val


def _gauss_weights(eaT3, valid, muA, ivA):
    cb = 40
    grid = NCH // cb
    return pl.pallas_call(
        _gw_body,
        grid=(grid,),
        in_specs=[
            pl.BlockSpec(memory_space=pltpu.SMEM),
            pl.BlockSpec(memory_space=pltpu.SMEM),
            pl.BlockSpec((D, cb, CHUNK), lambda i: (0, i, 0)),
            pl.BlockSpec((cb, CHUNK), lambda i: (i, 0)),
        ],
        out_specs=pl.BlockSpec((4, cb, 5, CHUNK), lambda i: (0, i, 0, 0)),
        out_shape=jax.ShapeDtypeStruct((4, NCH, 5, CHUNK), jnp.float32),
    )(muA, ivA, eaT3, valid)


def _lin_body(x_ref, w_ref, b_ref, y_ref, r_ref):
    z = jnp.dot(x_ref[...], w_ref[...], preferred_element_type=jnp.float32)
    z = z + b_ref[...]
    y_ref[...] = z[:, : K * H]
    r_ref[...] = z[:, K * H :]


def _lin(x, wcat, bcat):
    f = x.shape[1]
    rb = 1000
    grid = N // rb
    return pl.pallas_call(
        _lin_body,
        grid=(grid,),
        in_specs=[
            pl.BlockSpec((rb, f), lambda i: (i, 0)),
            pl.BlockSpec((f, K * H + H), lambda i: (0, 0)),
            pl.BlockSpec((1, K * H + H), lambda i: (0, 0)),
        ],
        out_specs=(
            pl.BlockSpec((rb, K * H), lambda i: (i, 0)),
            pl.BlockSpec((rb, H), lambda i: (i, 0)),
        ),
        out_shape=(
            jax.ShapeDtypeStruct((N, K * H), jnp.float32),
            jax.ShapeDtypeStruct((N, H), jnp.float32),
        ),
    )(x, wcat, bcat)


def _bcast16(vec, lane):
    idx = jnp.full((16, 1), lane, jnp.int32)
    dn = lax.GatherDimensionNumbers(
        offset_dims=(), collapsed_slice_dims=(0,), start_index_map=(0,)
    )
    return lax.gather(
        vec, idx, dn, (1,), mode=lax.GatherScatterMode.PROMISE_IN_BOUNDS
    )


def _sc_conv_body(y_hbm, src_hbm, dst_hbm, g_hbm, z_hbm, out_hbm,
                  idx_s, idx_d, gbuf, rows, msg, sem, acc):
    c = lax.axis_index("c")
    s = lax.axis_index("s")
    wid = s * 2 + c

    if True:
        @pl.when(s < 15)
        def _():
            pltpu.sync_copy(z_hbm, acc.at[pl.ds(s * RPT, RPT)])

        @pl.when(s == 15)
        def _():
            pltpu.sync_copy(z_hbm.at[pl.ds(0, RPT_LAST)],
                            acc.at[pl.ds(15 * RPT, RPT_LAST)])

        plsc.subcore_barrier()

        e0 = jnp.where(lax.iota(jnp.int32, 16) == 0, 1.0, 0.0).astype(jnp.float32)

        zv = jnp.zeros((16,), jnp.float32)
        for e in range(CHUNK):
            for j in range(5, 8):
                msg[e, pl.ds(j * 16, 16)] = zv

        def chunk_body(i, _):
            ch = wid * CPW + i
            base = ch * CHUNK
            pltpu.sync_copy(src_hbm.at[pl.ds(base, CHUNK)], idx_s)
            pltpu.sync_copy(dst_hbm.at[pl.ds(base, CHUNK)], idx_d)
            pltpu.sync_copy(g_hbm.at[ch], gbuf)
            pltpu.async_copy(y_hbm.at[idx_s], rows, sem).wait()

            def sub_body(sb, _):
                e00 = sb * 16
                gv0 = gbuf[0, pl.ds(e00, 16)]
                gv1 = gbuf[1, pl.ds(e00, 16)]
                gv2 = gbuf[2, pl.ds(e00, 16)]
                gv3 = gbuf[3, pl.ds(e00, 16)]
                vv = gbuf[4, pl.ds(e00, 16)]
                for b in range(16):
                    e = e00 + b
                    b0 = _bcast16(gv0, b)
                    b1 = _bcast16(gv1, b)
                    b2 = _bcast16(gv2, b)
                    b3 = _bcast16(gv3, b)
                    for j in range(4):
                        m = b0 * rows[e, pl.ds(j * 16, 16)]
                        m = m + b1 * rows[e, pl.ds(64 + j * 16, 16)]
                        m = m + b2 * rows[e, pl.ds(128 + j * 16, 16)]
                        m = m + b3 * rows[e, pl.ds(192 + j * 16, 16)]
                        msg[e, pl.ds(j * 16, 16)] = m
                    msg[e, pl.ds(64, 16)] = _bcast16(vv, b) * e0
                return 0

            lax.fori_loop(0, 8, sub_body, 0)
            pltpu.sync_copy(msg, acc.at[idx_d], add=True)
            return 0

        lax.fori_loop(0, CPW, chunk_body, 0)
        plsc.subcore_barrier()

        @pl.when(s < 15)
        def _():
            pltpu.sync_copy(acc.at[pl.ds(s * RPT, RPT)],
                            out_hbm.at[c, pl.ds(s * RPT, RPT)])

        @pl.when(s == 15)
        def _():
            pltpu.sync_copy(acc.at[pl.ds(15 * RPT, RPT_LAST)],
                            out_hbm.at[c, pl.ds(15 * RPT, RPT_LAST)])


def _sc_conv(y, srcp, dstp, gc, zrows):
    mesh = plsc.VectorSubcoreMesh(core_axis_name="c", subcore_axis_name="s")
    return pl.kernel(
        _sc_conv_body,
        out_type=jax.ShapeDtypeStruct((2, N, AW), jnp.float32),
        mesh=mesh,
        scratch_types=[
            pltpu.VMEM((CHUNK,), jnp.int32),
            pltpu.VMEM((CHUNK,), jnp.int32),
            pltpu.VMEM((5, CHUNK), jnp.float32),
            pltpu.VMEM((CHUNK, K * H), jnp.float32),
            pltpu.VMEM((CHUNK, AW), jnp.float32),
            pltpu.SemaphoreType.DMA,
            pltpu.VMEM_SHARED((N, AW), jnp.float32),
        ],
    )(y, srcp, dstp, gc, zrows)


def _fin_body(p_ref, r_ref, w_ref, b_ref, h_ref, y_ref, rn_ref):
    sacc = p_ref[0] + p_ref[1]
    cnt = jnp.maximum(sacc[:, H : H + 1], 1.0)
    h = _leaky(sacc[:, :H] / cnt + r_ref[...])
    h_ref[...] = h
    z = jnp.dot(h, w_ref[...], preferred_element_type=jnp.float32) + b_ref[...]
    y_ref[...] = z[:, : K * H]
    rn_ref[...] = z[:, K * H :]


def _fin(p, r, wcat, bcat):
    rb = 1000
    return pl.pallas_call(
        _fin_body,
        grid=(N // rb,),
        in_specs=[
            pl.BlockSpec((2, rb, AW), lambda i: (0, i, 0)),
            pl.BlockSpec((rb, H), lambda i: (i, 0)),
            pl.BlockSpec((H, K * H + H), lambda i: (0, 0)),
            pl.BlockSpec((1, K * H + H), lambda i: (0, 0)),
        ],
        out_specs=(
            pl.BlockSpec((rb, H), lambda i: (i, 0)),
            pl.BlockSpec((rb, K * H), lambda i: (i, 0)),
            pl.BlockSpec((rb, H), lambda i: (i, 0)),
        ),
        out_shape=(
            jax.ShapeDtypeStruct((N, H), jnp.float32),
            jax.ShapeDtypeStruct((N, K * H), jnp.float32),
            jax.ShapeDtypeStruct((N, H), jnp.float32),
        ),
    )(p, r, wcat, bcat)


def _fin_res_body(p_ref, r_ref, hp_ref, w_ref, b_ref, h_ref, h3_ref, y_ref, rn_ref):
    sacc = p_ref[0] + p_ref[1]
    cnt = jnp.maximum(sacc[:, H : H + 1], 1.0)
    h = _leaky(sacc[:, :H] / cnt + r_ref[...])
    h_ref[...] = h
    h3 = hp_ref[...] + h
    h3_ref[...] = h3
    z = jnp.dot(h3, w_ref[...], preferred_element_type=jnp.float32) + b_ref[...]
    y_ref[...] = z[:, : K * H]
    rn_ref[...] = z[:, K * H :]


def _fin_res(p, r, hprev, wcat, bcat):
    rb = 1000
    return pl.pallas_call(
        _fin_res_body,
        grid=(N // rb,),
        in_specs=[
            pl.BlockSpec((2, rb, AW), lambda i: (0, i, 0)),
            pl.BlockSpec((rb, H), lambda i: (i, 0)),
            pl.BlockSpec((rb, H), lambda i: (i, 0)),
            pl.BlockSpec((H, K * H + H), lambda i: (0, 0)),
            pl.BlockSpec((1, K * H + H), lambda i: (0, 0)),
        ],
        out_specs=(
            pl.BlockSpec((rb, H), lambda i: (i, 0)),
            pl.BlockSpec((rb, H), lambda i: (i, 0)),
            pl.BlockSpec((rb, K * H), lambda i: (i, 0)),
            pl.BlockSpec((rb, H), lambda i: (i, 0)),
        ),
        out_shape=(
            jax.ShapeDtypeStruct((N, H), jnp.float32),
            jax.ShapeDtypeStruct((N, H), jnp.float32),
            jax.ShapeDtypeStruct((N, K * H), jnp.float32),
            jax.ShapeDtypeStruct((N, H), jnp.float32),
        ),
    )(p, r, hprev, wcat, bcat)


def _head_body(bat_s, p_ref, r_ref, h1_ref, h2_ref, h3_ref, bat_ref,
               w1_ref, b1_ref, sc_ref, sh_ref, w2_ref, b2_ref,
               out_ref, acc):
    i = pl.program_id(0)
    rb = h1_ref.shape[0]

    @pl.when(i == 0)
    def _():
        acc[...] = jnp.full((G, 4 * H), -jnp.inf, jnp.float32)

    sacc = p_ref[0] + p_ref[1]
    cnt = jnp.maximum(sacc[:, H : H + 1], 1.0)
    h4 = _leaky(sacc[:, :H] / cnt + r_ref[...])
    hc = jnp.concatenate([h4, h1_ref[...], h2_ref[...], h3_ref[...]], axis=1)

    lo = bat_s[i * rb]
    hi = bat_s[i * rb + rb - 1]
    bat = bat_ref[...]
    for g in range(G):
        @pl.when((g >= lo) & (g <= hi))
        def _():
            m = bat == g
            masked = jnp.where(m, hc, -jnp.inf)
            acc[g, :] = jnp.maximum(acc[g, :], jnp.max(masked, axis=0))

    @pl.when(i == pl.num_programs(0) - 1)
    def _():
        pooled = acc[...]
        pooled = jnp.where(jnp.isfinite(pooled), pooled, 0.0)
        z = jnp.dot(pooled, w1_ref[...], preferred_element_type=jnp.float32)
        z = z + b1_ref[...]
        z = z * sc_ref[...] + sh_ref[...]
        z = jnp.maximum(z, 0.0)
        out_ref[...] = (
            jnp.dot(z, w2_ref[...], preferred_element_type=jnp.float32)
            + b2_ref[...]
        )


def _head(p, r3, h1, h2, h3, batch2, mW1, mb1, scale, shift, mW2, mb2):
    rb = 1000
    return pl.pallas_call(
        _head_body,
        grid=(N // rb,),
        in_specs=[
            pl.BlockSpec(memory_space=pltpu.SMEM),
            pl.BlockSpec((2, rb, AW), lambda i: (0, i, 0)),
            pl.BlockSpec((rb, H), lambda i: (i, 0)),
            pl.BlockSpec((rb, H), lambda i: (i, 0)),
            pl.BlockSpec((rb, H), lambda i: (i, 0)),
            pl.BlockSpec((rb, H), lambda i: (i, 0)),
            pl.BlockSpec((rb, 1), lambda i: (i, 0)),
            pl.BlockSpec((4 * H, H), lambda i: (0, 0)),
            pl.BlockSpec((1, H), lambda i: (0, 0)),
            pl.BlockSpec((1, H), lambda i: (0, 0)),
            pl.BlockSpec((1, H), lambda i: (0, 0)),
            pl.BlockSpec((H, C), lambda i: (0, 0)),
            pl.BlockSpec((1, C), lambda i: (0, 0)),
        ],
        out_specs=pl.BlockSpec((G, C), lambda i: (0, 0)),
        out_shape=jax.ShapeDtypeStruct((G, C), jnp.float32),
        scratch_shapes=[pltpu.VMEM((G, 4 * H), jnp.float32)],
    )(batch2[:, 0], p, r3, h1, h2, h3, batch2, mW1, mb1, scale, shift, mW2, mb2)


def kernel(x, edge_index, batch, edge_attr,
           Wg0, Wr0, b0, mu0, sg0, Wg1, Wr1, b1, mu1, sg1,
           Wg2, Wr2, b2, mu2, sg2, Wg3, Wr3, b3, mu3, sg3,
           mW1, mb1, bn_g, bn_b, bn_m, bn_v, mW2, mb2):
    pad = E_PAD - E
    srcp = jnp.pad(edge_index[0], (0, pad))
    dstp = jnp.pad(edge_index[1], (0, pad))
    eaT3 = jnp.pad(edge_attr, ((0, pad), (0, 0))).T.reshape(D, NCH, CHUNK)
    valid = jnp.pad(jnp.ones((E,), jnp.float32), (0, pad)).reshape(NCH, CHUNK)

    muA = jnp.stack([mu0, mu1, mu2, mu3])
    ivA = 1.0 / (EPS + jnp.stack([sg0, sg1, sg2, sg3]) ** 2)
    gc = _gauss_weights(eaT3, valid, muA, ivA)

    def wcat(Wg, Wr, b):
        return (jnp.concatenate([Wg, Wr], axis=1),
                jnp.concatenate([jnp.zeros((K * H,), jnp.float32), b])[None, :])

    zrows = jnp.zeros((RPT, AW), jnp.float32)

    w0, bc0 = wcat(Wg0, Wr0, b0)
    y0, r0 = _lin(x, w0, bc0)
    p0 = _sc_conv(y0, srcp, dstp, gc[0], zrows)

    w1, bc1 = wcat(Wg1, Wr1, b1)
    h, y1, r1 = _fin(p0, r0, w1, bc1)
    p1 = _sc_conv(y1, srcp, dstp, gc[1], zrows)

    w2, bc2 = wcat(Wg2, Wr2, b2)
    h1, y2, r2 = _fin(p1, r1, w2, bc2)
    p2 = _sc_conv(y2, srcp, dstp, gc[2], zrows)

    w3, bc3 = wcat(Wg3, Wr3, b3)
    h2, h3, y3, r3 = _fin_res(p2, r2, h, w3, bc3)
    p3 = _sc_conv(y3, srcp, dstp, gc[3], zrows)

    scale = bn_g / jnp.sqrt(bn_v + 1e-5)
    shift = bn_b - bn_m * scale
    return _head(p3, r3, h1, h2, h3, batch[:, None].astype(jnp.int32),
                 mW1, mb1[None, :], scale[None, :], shift[None, :],
                 mW2, mb2[None, :])


# speedup vs baseline: 3.1002x; 1.5435x over previous
import functools

import jax
import jax.numpy as jnp
from jax import lax
from jax.experimental import pallas as pl
from jax.experimental.pallas import tpu as pltpu
from jax.experimental.pallas import tpu_sc as plsc

N = 10000
E = 160000
K = 4
D = 4
H = 64
F_IN = 128
C = 10
G = 64
EPS = 1e-15
ALPHA = 0.01

CHUNK = 64
NW = 32
NCH = 2560
E_PAD = NCH * CHUNK
CPW = NCH // NW
RPT = 632
RPT_LAST = N - 15 * RPT
AW = 128


def _leaky(v):
    return jnp.where(v >= 0, v, ALPHA * v)


def _gw_body(mu_ref, iv_ref, ea_ref, val_ref, out_ref):
    val = val_ref[...]
    for l in range(4):
        for k in range(K):
            acc = jnp.zeros_like(val)
            for d in range(D):
                t = ea_ref[d] - mu_ref[l, k, d]
                acc = acc + t * t * iv_ref[l, k, d]
            out_ref[l, :, k, :] = jnp.exp(-0.5 * acc) * val
        out_ref[l, :, 4, :] = val


def _gauss_weights(eaT3, valid, muA, ivA):
    cb = 40
    grid = NCH // cb
    return pl.pallas_call(
        _gw_body,
        grid=(grid,),
        in_specs=[
            pl.BlockSpec(memory_space=pltpu.SMEM),
            pl.BlockSpec(memory_space=pltpu.SMEM),
            pl.BlockSpec((D, cb, CHUNK), lambda i: (0, i, 0)),
            pl.BlockSpec((cb, CHUNK), lambda i: (i, 0)),
        ],
        out_specs=pl.BlockSpec((4, cb, 5, CHUNK), lambda i: (0, i, 0, 0)),
        out_shape=jax.ShapeDtypeStruct((4, NCH, 5, CHUNK), jnp.float32),
    )(muA, ivA, eaT3, valid)


def _lin_body(x_ref, w_ref, b_ref, y_ref, r_ref):
    z = jnp.dot(x_ref[...], w_ref[...], preferred_element_type=jnp.float32)
    z = z + b_ref[...]
    y_ref[...] = z[:, : K * H]
    r_ref[...] = z[:, K * H :]


def _lin(x, wcat, bcat):
    f = x.shape[1]
    rb = 1000
    grid = N // rb
    return pl.pallas_call(
        _lin_body,
        grid=(grid,),
        in_specs=[
            pl.BlockSpec((rb, f), lambda i: (i, 0)),
            pl.BlockSpec((f, K * H + H), lambda i: (0, 0)),
            pl.BlockSpec((1, K * H + H), lambda i: (0, 0)),
        ],
        out_specs=(
            pl.BlockSpec((rb, K * H), lambda i: (i, 0)),
            pl.BlockSpec((rb, H), lambda i: (i, 0)),
        ),
        out_shape=(
            jax.ShapeDtypeStruct((N, K * H), jnp.float32),
            jax.ShapeDtypeStruct((N, H), jnp.float32),
        ),
    )(x, wcat, bcat)


def _bcast16(vec, lane):
    idx = jnp.full((16, 1), lane, jnp.int32)
    dn = lax.GatherDimensionNumbers(
        offset_dims=(), collapsed_slice_dims=(0,), start_index_map=(0,)
    )
    return lax.gather(
        vec, idx, dn, (1,), mode=lax.GatherScatterMode.PROMISE_IN_BOUNDS
    )


def _sc_conv_body(y_hbm, src_hbm, dst_hbm, g_hbm, z_hbm, out_hbm,
                  idx_s, idx_d, scidx, gbuf, msg,
                  gsem0, gsem1, ssem0, ssem1, wsem, acc):
    c = lax.axis_index("c")
    s = lax.axis_index("s")
    wid = s * 2 + c
    ch0 = wid * CPW

    def _run(rows):
        @pl.when(s < 15)
        def _():
            pltpu.sync_copy(z_hbm, acc.at[pl.ds(s * RPT, RPT)])

        @pl.when(s == 15)
        def _():
            pltpu.sync_copy(z_hbm.at[pl.ds(0, RPT_LAST)],
                            acc.at[pl.ds(15 * RPT, RPT_LAST)])

        plsc.subcore_barrier()

        e0 = jnp.where(lax.iota(jnp.int32, 16) == 0, 1.0, 0.0).astype(jnp.float32)

        zv = jnp.zeros((16,), jnp.float32)
        for e in range(CHUNK):
            for j in range(5, 8):
                msg[e, pl.ds(j * 16, 16)] = zv

        gsem = (gsem0, gsem1)
        ssem = (ssem0, ssem1)

        def stage(i, p):
            pltpu.async_copy(src_hbm.at[ch0 + i], idx_s.at[p], ssem[p])
            pltpu.async_copy(dst_hbm.at[ch0 + i], idx_d.at[p], ssem[p])
            pltpu.async_copy(g_hbm.at[ch0 + i], gbuf.at[p], ssem[p])

        def stage_wait(p):
            pltpu.make_async_copy(src_hbm.at[ch0], idx_s.at[p], ssem[p]).wait()
            pltpu.make_async_copy(dst_hbm.at[ch0], idx_d.at[p], ssem[p]).wait()
            pltpu.make_async_copy(g_hbm.at[ch0], gbuf.at[p], ssem[p]).wait()

        def gather(p):
            pltpu.async_copy(y_hbm.at[idx_s.at[p]], rows.at[p], gsem[p])

        def gather_wait(p):
            pltpu.make_async_copy(y_hbm.at[idx_s.at[p]], rows.at[p],
                                  gsem[p]).wait()

        def scatter_wait(p):
            pltpu.make_async_copy(msg, acc.at[scidx.at[p]], wsem).wait()

        def step(i, p):
            @pl.when(i + 1 < CPW)
            def _():
                stage_wait(1 - p)
                gather(1 - p)

            gather_wait(p)

            @pl.when(i >= 1)
            def _():
                scatter_wait(1 - p)

            def sub_body(sb, _):
                e00 = sb * 16
                gv0 = gbuf[p, 0, pl.ds(e00, 16)]
                gv1 = gbuf[p, 1, pl.ds(e00, 16)]
                gv2 = gbuf[p, 2, pl.ds(e00, 16)]
                gv3 = gbuf[p, 3, pl.ds(e00, 16)]
                vv = gbuf[p, 4, pl.ds(e00, 16)]
                for b in range(16):
                    e = e00 + b
                    b0 = _bcast16(gv0, b)
                    b1 = _bcast16(gv1, b)
                    b2 = _bcast16(gv2, b)
                    b3 = _bcast16(gv3, b)
                    for j in range(4):
                        m = b0 * rows[p, e, pl.ds(j * 16, 16)]
                        m = m + b1 * rows[p, e, pl.ds(64 + j * 16, 16)]
                        m = m + b2 * rows[p, e, pl.ds(128 + j * 16, 16)]
                        m = m + b3 * rows[p, e, pl.ds(192 + j * 16, 16)]
                        msg[e, pl.ds(j * 16, 16)] = m
                    msg[e, pl.ds(64, 16)] = _bcast16(vv, b) * e0
                return 0

            lax.fori_loop(0, CHUNK // 16, sub_body, 0)
            for q in range(CHUNK // 16):
                scidx[p, pl.ds(q * 16, 16)] = idx_d[p, pl.ds(q * 16, 16)]
            pltpu.async_copy(msg, acc.at[scidx.at[p]], wsem, add=True)

            @pl.when(i + 2 < CPW)
            def _():
                stage(i + 2, p)

        stage(0, 0)
        stage_wait(0)
        gather(0)
        stage(1, 1)

        def pair_body(it, _):
            i = it * 2
            step(i, 0)
            step(i + 1, 1)
            return 0

        lax.fori_loop(0, CPW // 2, pair_body, 0)
        scatter_wait(1)
        plsc.subcore_barrier()

    pl.run_scoped(_run, pltpu.VMEM((2, CHUNK, K * H), jnp.float32))

    @pl.when(s < 15)
    def _():
        pltpu.sync_copy(acc.at[pl.ds(s * RPT, RPT)],
                        out_hbm.at[c, pl.ds(s * RPT, RPT)])

    @pl.when(s == 15)
    def _():
        pltpu.sync_copy(acc.at[pl.ds(15 * RPT, RPT_LAST)],
                        out_hbm.at[c, pl.ds(15 * RPT, RPT_LAST)])


def _sc_conv(y, srcp, dstp, gc, zrows):
    mesh = plsc.VectorSubcoreMesh(core_axis_name="c", subcore_axis_name="s")
    return pl.kernel(
        _sc_conv_body,
        out_type=jax.ShapeDtypeStruct((2, N, AW), jnp.float32),
        mesh=mesh,
        scratch_types=[
            pltpu.VMEM((2, CHUNK), jnp.int32),
            pltpu.VMEM((2, CHUNK), jnp.int32),
            pltpu.VMEM((2, CHUNK), jnp.int32),
            pltpu.VMEM((2, 5, CHUNK), jnp.float32),
            pltpu.VMEM((CHUNK, AW), jnp.float32),
            pltpu.SemaphoreType.DMA,
            pltpu.SemaphoreType.DMA,
            pltpu.SemaphoreType.DMA,
            pltpu.SemaphoreType.DMA,
            pltpu.SemaphoreType.DMA,
            pltpu.VMEM_SHARED((N, AW), jnp.float32),
        ],
    )(y, srcp, dstp, gc, zrows)


def _fin_body(p_ref, r_ref, w_ref, b_ref, h_ref, y_ref, rn_ref):
    sacc = p_ref[0] + p_ref[1]
    cnt = jnp.maximum(sacc[:, H : H + 1], 1.0)
    h = _leaky(sacc[:, :H] / cnt + r_ref[...])
    h_ref[...] = h
    z = jnp.dot(h, w_ref[...], preferred_element_type=jnp.float32) + b_ref[...]
    y_ref[...] = z[:, : K * H]
    rn_ref[...] = z[:, K * H :]


def _fin(p, r, wcat, bcat):
    rb = 1000
    return pl.pallas_call(
        _fin_body,
        grid=(N // rb,),
        in_specs=[
            pl.BlockSpec((2, rb, AW), lambda i: (0, i, 0)),
            pl.BlockSpec((rb, H), lambda i: (i, 0)),
            pl.BlockSpec((H, K * H + H), lambda i: (0, 0)),
            pl.BlockSpec((1, K * H + H), lambda i: (0, 0)),
        ],
        out_specs=(
            pl.BlockSpec((rb, H), lambda i: (i, 0)),
            pl.BlockSpec((rb, K * H), lambda i: (i, 0)),
            pl.BlockSpec((rb, H), lambda i: (i, 0)),
        ),
        out_shape=(
            jax.ShapeDtypeStruct((N, H), jnp.float32),
            jax.ShapeDtypeStruct((N, K * H), jnp.float32),
            jax.ShapeDtypeStruct((N, H), jnp.float32),
        ),
    )(p, r, wcat, bcat)


def _fin_res_body(p_ref, r_ref, hp_ref, w_ref, b_ref, h_ref, h3_ref, y_ref, rn_ref):
    sacc = p_ref[0] + p_ref[1]
    cnt = jnp.maximum(sacc[:, H : H + 1], 1.0)
    h = _leaky(sacc[:, :H] / cnt + r_ref[...])
    h_ref[...] = h
    h3 = hp_ref[...] + h
    h3_ref[...] = h3
    z = jnp.dot(h3, w_ref[...], preferred_element_type=jnp.float32) + b_ref[...]
    y_ref[...] = z[:, : K * H]
    rn_ref[...] = z[:, K * H :]


def _fin_res(p, r, hprev, wcat, bcat):
    rb = 1000
    return pl.pallas_call(
        _fin_res_body,
        grid=(N // rb,),
        in_specs=[
            pl.BlockSpec((2, rb, AW), lambda i: (0, i, 0)),
            pl.BlockSpec((rb, H), lambda i: (i, 0)),
            pl.BlockSpec((rb, H), lambda i: (i, 0)),
            pl.BlockSpec((H, K * H + H), lambda i: (0, 0)),
            pl.BlockSpec((1, K * H + H), lambda i: (0, 0)),
        ],
        out_specs=(
            pl.BlockSpec((rb, H), lambda i: (i, 0)),
            pl.BlockSpec((rb, H), lambda i: (i, 0)),
            pl.BlockSpec((rb, K * H), lambda i: (i, 0)),
            pl.BlockSpec((rb, H), lambda i: (i, 0)),
        ),
        out_shape=(
            jax.ShapeDtypeStruct((N, H), jnp.float32),
            jax.ShapeDtypeStruct((N, H), jnp.float32),
            jax.ShapeDtypeStruct((N, K * H), jnp.float32),
            jax.ShapeDtypeStruct((N, H), jnp.float32),
        ),
    )(p, r, hprev, wcat, bcat)


def _head_body(bat_s, p_ref, r_ref, h1_ref, h2_ref, h3_ref, bat_ref,
               w1_ref, b1_ref, sc_ref, sh_ref, w2_ref, b2_ref,
               out_ref, acc):
    i = pl.program_id(0)
    rb = h1_ref.shape[0]

    @pl.when(i == 0)
    def _():
        acc[...] = jnp.full((G, 4 * H), -jnp.inf, jnp.float32)

    sacc = p_ref[0] + p_ref[1]
    cnt = jnp.maximum(sacc[:, H : H + 1], 1.0)
    h4 = _leaky(sacc[:, :H] / cnt + r_ref[...])
    hc = jnp.concatenate([h4, h1_ref[...], h2_ref[...], h3_ref[...]], axis=1)

    lo = bat_s[i * rb]
    hi = bat_s[i * rb + rb - 1]
    bat = bat_ref[...]
    for g in range(G):
        @pl.when((g >= lo) & (g <= hi))
        def _():
            m = bat == g
            masked = jnp.where(m, hc, -jnp.inf)
            acc[g, :] = jnp.maximum(acc[g, :], jnp.max(masked, axis=0))

    @pl.when(i == pl.num_programs(0) - 1)
    def _():
        pooled = acc[...]
        pooled = jnp.where(jnp.isfinite(pooled), pooled, 0.0)
        z = jnp.dot(pooled, w1_ref[...], preferred_element_type=jnp.float32)
        z = z + b1_ref[...]
        z = z * sc_ref[...] + sh_ref[...]
        z = jnp.maximum(z, 0.0)
        out_ref[...] = (
            jnp.dot(z, w2_ref[...], preferred_element_type=jnp.float32)
            + b2_ref[...]
        )


def _head(p, r3, h1, h2, h3, batch2, mW1, mb1, scale, shift, mW2, mb2):
    rb = 1000
    return pl.pallas_call(
        _head_body,
        grid=(N // rb,),
        in_specs=[
            pl.BlockSpec(memory_space=pltpu.SMEM),
            pl.BlockSpec((2, rb, AW), lambda i: (0, i, 0)),
            pl.BlockSpec((rb, H), lambda i: (i, 0)),
            pl.BlockSpec((rb, H), lambda i: (i, 0)),
            pl.BlockSpec((rb, H), lambda i: (i, 0)),
            pl.BlockSpec((rb, H), lambda i: (i, 0)),
            pl.BlockSpec((rb, 1), lambda i: (i, 0)),
            pl.BlockSpec((4 * H, H), lambda i: (0, 0)),
            pl.BlockSpec((1, H), lambda i: (0, 0)),
            pl.BlockSpec((1, H), lambda i: (0, 0)),
            pl.BlockSpec((1, H), lambda i: (0, 0)),
            pl.BlockSpec((H, C), lambda i: (0, 0)),
            pl.BlockSpec((1, C), lambda i: (0, 0)),
        ],
        out_specs=pl.BlockSpec((G, C), lambda i: (0, 0)),
        out_shape=jax.ShapeDtypeStruct((G, C), jnp.float32),
        scratch_shapes=[pltpu.VMEM((G, 4 * H), jnp.float32)],
    )(batch2[:, 0], p, r3, h1, h2, h3, batch2, mW1, mb1, scale, shift, mW2, mb2)


def kernel(x, edge_index, batch, edge_attr,
           Wg0, Wr0, b0, mu0, sg0, Wg1, Wr1, b1, mu1, sg1,
           Wg2, Wr2, b2, mu2, sg2, Wg3, Wr3, b3, mu3, sg3,
           mW1, mb1, bn_g, bn_b, bn_m, bn_v, mW2, mb2):
    pad = E_PAD - E
    srcp = jnp.pad(edge_index[0], (0, pad)).reshape(NCH, CHUNK)
    dstp = jnp.pad(edge_index[1], (0, pad)).reshape(NCH, CHUNK)
    eaT3 = jnp.pad(edge_attr, ((0, pad), (0, 0))).T.reshape(D, NCH, CHUNK)
    valid = jnp.pad(jnp.ones((E,), jnp.float32), (0, pad)).reshape(NCH, CHUNK)

    muA = jnp.stack([mu0, mu1, mu2, mu3])
    ivA = 1.0 / (EPS + jnp.stack([sg0, sg1, sg2, sg3]) ** 2)
    gc = _gauss_weights(eaT3, valid, muA, ivA)

    def wcat(Wg, Wr, b):
        return (jnp.concatenate([Wg, Wr], axis=1),
                jnp.concatenate([jnp.zeros((K * H,), jnp.float32), b])[None, :])

    zrows = jnp.zeros((RPT, AW), jnp.float32)

    w0, bc0 = wcat(Wg0, Wr0, b0)
    y0, r0 = _lin(x, w0, bc0)
    p0 = _sc_conv(y0, srcp, dstp, gc[0], zrows)

    w1, bc1 = wcat(Wg1, Wr1, b1)
    h, y1, r1 = _fin(p0, r0, w1, bc1)
    p1 = _sc_conv(y1, srcp, dstp, gc[1], zrows)

    w2, bc2 = wcat(Wg2, Wr2, b2)
    h1, y2, r2 = _fin(p1, r1, w2, bc2)
    p2 = _sc_conv(y2, srcp, dstp, gc[2], zrows)

    w3, bc3 = wcat(Wg3, Wr3, b3)
    h2, h3, y3, r3 = _fin_res(p2, r2, h, w3, bc3)
    p3 = _sc_conv(y3, srcp, dstp, gc[3], zrows)

    scale = bn_g / jnp.sqrt(bn_v + 1e-5)
    shift = bn_b - bn_m * scale
    return _head(p3, r3, h1, h2, h3, batch[:, None].astype(jnp.int32),
                 mW1, mb1[None, :], scale[None, :], shift[None, :],
                 mW2, mb2[None, :])
